# baseline (speedup 1.0000x reference)
import jax
import jax.numpy as jnp
from jax.experimental import pallas as pl
from jax.experimental.pallas import tpu as pltpu

_TAU = 0.07


def _fused(mask_ref, fq_ref, fk_ref, out_ref, sq_acc, sk_acc, cnt_acc):
    i = pl.program_id(0)
    nb = pl.num_programs(0)

    mnum, hh, ww = mask_ref.shape[1:]
    c = fq_ref.shape[1]
    m2 = mask_ref[0].astype(jnp.float32).reshape(mnum, hh * ww)
    fq2 = fq_ref[0].reshape(c, hh * ww)
    fk2 = fk_ref[0].reshape(c, hh * ww)
    dn = (((1,), (1,)), ((), ()))
    sq = jax.lax.dot_general(m2, fq2, dn,
                             preferred_element_type=jnp.float32)
    sk = jax.lax.dot_general(m2, fk2, dn,
                             preferred_element_type=jnp.float32)
    cnt = jnp.sum(m2, axis=1, keepdims=True)

    sq_acc[i] = sq
    sk_acc[i] = sk
    cnt_acc[i] = cnt

    @pl.when(i == nb - 1)
    def _epilogue():
        nbatch, mnum, c = sq_acc.shape
        n = nbatch * mnum
        cntv = jnp.maximum(cnt_acc[...].reshape(n, 1), 1.0)
        mq = sq_acc[...].reshape(n, c) / cntv
        mk = sk_acc[...].reshape(n, c) / cntv
        pad = (mk[:, 0:1] != 0).astype(jnp.float32)
        nq = mq / jnp.maximum(
            jnp.sqrt(jnp.sum(mq * mq, axis=-1, keepdims=True)), 1e-12)
        nkv = mk / jnp.maximum(
            jnp.sqrt(jnp.sum(mk * mk, axis=-1, keepdims=True)), 1e-12)
        dn2 = (((1,), (1,)), ((), ()))
        rows = jax.lax.dot_general(nkv, nq, dn2,
                                   preferred_element_type=jnp.float32) / _TAU
        mx = jnp.max(rows, axis=-1, keepdims=True)
        lse = jnp.log(jnp.sum(jnp.exp(rows - mx), axis=-1,
                              keepdims=True)) + mx
        ii = jax.lax.broadcasted_iota(jnp.int32, (n, n), 0)
        jj = jax.lax.broadcasted_iota(jnp.int32, (n, n), 1)
        diag = jnp.sum(jnp.where(ii == jj, rows, 0.0), axis=-1,
                       keepdims=True)
        ce = lse - diag
        num = jnp.sum(ce * pad)
        den = jnp.maximum(jnp.sum(pad), 1.0)
        out_ref[...] = jnp.reshape(num / den, (1, 1))


def kernel(features_q, features_k, pos_region_ranges):
    b, c, h, w = features_q.shape
    mnum = pos_region_ranges.shape[1]

    loss = pl.pallas_call(
        _fused,
        grid=(b,),
        in_specs=[
            pl.BlockSpec((1, mnum, h, w), lambda i: (i, 0, 0, 0)),
            pl.BlockSpec((1, c, h, w), lambda i: (i, 0, 0, 0)),
            pl.BlockSpec((1, c, h, w), lambda i: (i, 0, 0, 0)),
        ],
        out_specs=pl.BlockSpec((1, 1), lambda i: (0, 0)),
        out_shape=jax.ShapeDtypeStruct((1, 1), jnp.float32),
        scratch_shapes=[
            pltpu.VMEM((b, mnum, c), jnp.float32),
            pltpu.VMEM((b, mnum, c), jnp.float32),
            pltpu.VMEM((b, mnum, 1), jnp.float32),
        ],
    )(pos_region_ranges, features_q, features_k)
    return loss[0, 0]
